# baseline (speedup 1.0000x reference)
import functools

import jax
import jax.numpy as jnp
from jax import lax
from jax.experimental import pallas as pl
from jax.experimental.pallas import tpu as pltpu
from jax.experimental.pallas import tpu_sc as plsc

VOCAB = 30522
HIDDEN = 768
BATCH = 64
SEQ = 512
EPS = 1e-12

NC = 2
NS = 16
NW = NC * NS
PPW = SEQ // NW
NJ = HIDDEN // 16


def _lanesum16(v):
    idx = lax.iota(jnp.int32, 16)
    dnums = lax.GatherDimensionNumbers(
        offset_dims=(), collapsed_slice_dims=(0,), start_index_map=(0,))
    for sh in (8, 4, 2, 1):
        perm = (idx ^ sh)[:, None]
        v = v + lax.gather(v, perm, dimension_numbers=dnums, slice_sizes=(1,),
                           unique_indices=True,
                           mode=lax.GatherScatterMode.PROMISE_IN_BOUNDS)
    return v


def _rsqrt16(v):
    x2 = v * 0.5
    i = lax.bitcast_convert_type(v, jnp.int32)
    i = jnp.int32(0x5F3759DF) - (i >> 1)
    y = lax.bitcast_convert_type(i, jnp.float32)
    y = y * (1.5 - x2 * y * y)
    y = y * (1.5 - x2 * y * y)
    y = y * (1.5 - x2 * y * y)
    y = y * (1.5 - x2 * y * y)
    return y


def _sc_body(ids_w, wemb, pemb, gamma, beta, out,
             idx_v, pos_v, g_v, b_v,
             in_a, in_b, out_a, out_b,
             gsem_a, gsem_b, ssem_a, ssem_b):
    c = lax.axis_index("c")
    s = lax.axis_index("s")
    w = s * NC + c

    pltpu.sync_copy(ids_w.at[w], idx_v)
    pltpu.sync_copy(pemb.at[pl.ds(w * PPW, PPW)], pos_v)
    pltpu.sync_copy(gamma, g_v)
    pltpu.sync_copy(beta, b_v)

    def gather(b, buf, sem):
        pltpu.async_copy(wemb.at[idx_v.at[pl.ds(b * PPW, PPW)]], buf, sem)

    def compute(xin, xout):
        @plsc.parallel_loop(0, PPW, 1, unroll=2)
        def row_body(r):
            acc = [jnp.zeros((16,), jnp.float32) for _ in range(4)]
            accq = [jnp.zeros((16,), jnp.float32) for _ in range(4)]
            for j in range(NJ):
                sl = pl.ds(j * 16, 16)
                x = xin[r, sl] + pos_v[r, sl]
                xout[r, sl] = x
                acc[j % 4] = acc[j % 4] + x
                accq[j % 4] = accq[j % 4] + x * x
            sum_v = (acc[0] + acc[1]) + (acc[2] + acc[3])
            sq_v = (accq[0] + accq[1]) + (accq[2] + accq[3])
            mean_v = _lanesum16(sum_v) * (1.0 / HIDDEN)
            msq_v = _lanesum16(sq_v) * (1.0 / HIDDEN)
            var_v = msq_v - mean_v * mean_v
            inv_v = _rsqrt16(var_v + EPS)
            for j in range(NJ):
                sl = pl.ds(j * 16, 16)
                xout[r, sl] = (xout[r, sl] - mean_v) * inv_v * g_v[sl] + b_v[sl]

    def halfstep(i, b, xin, xout, gsem, ssem):
        @pl.when(i > 0)
        def _():
            pltpu.make_async_copy(xout, out.at[0, pl.ds(w * PPW, PPW)], ssem).wait()

        pltpu.make_async_copy(wemb.at[pl.ds(0, PPW)], xin, gsem).wait()
        compute(xin, xout)

        @pl.when(b + 2 < BATCH)
        def _():
            gather(b + 2, xin, gsem)

        pltpu.async_copy(xout, out.at[b, pl.ds(w * PPW, PPW)], ssem)

    gather(0, in_a, gsem_a)
    gather(1, in_b, gsem_b)

    def loop_body(i, carry):
        halfstep(i, 2 * i, in_a, out_a, gsem_a, ssem_a)
        halfstep(i, 2 * i + 1, in_b, out_b, gsem_b, ssem_b)
        return carry

    lax.fori_loop(0, BATCH // 2, loop_body, 0)

    pltpu.make_async_copy(out_a, out.at[0, pl.ds(w * PPW, PPW)], ssem_a).wait()
    pltpu.make_async_copy(out_b, out.at[0, pl.ds(w * PPW, PPW)], ssem_b).wait()


@functools.partial(jax.jit, static_argnames=())
def _run(ids_w, word_emb, pos_emb, ln_gamma, ln_beta):
    kern = pl.kernel(
        _sc_body,
        out_type=jax.ShapeDtypeStruct((BATCH, SEQ, HIDDEN), jnp.float32),
        mesh=plsc.VectorSubcoreMesh(core_axis_name="c", subcore_axis_name="s"),
        scratch_types=[
            pltpu.VMEM((BATCH * PPW,), jnp.int32),
            pltpu.VMEM((PPW, HIDDEN), jnp.float32),
            pltpu.VMEM((HIDDEN,), jnp.float32),
            pltpu.VMEM((HIDDEN,), jnp.float32),
            pltpu.VMEM((PPW, HIDDEN), jnp.float32),
            pltpu.VMEM((PPW, HIDDEN), jnp.float32),
            pltpu.VMEM((PPW, HIDDEN), jnp.float32),
            pltpu.VMEM((PPW, HIDDEN), jnp.float32),
            pltpu.SemaphoreType.DMA,
            pltpu.SemaphoreType.DMA,
            pltpu.SemaphoreType.DMA,
            pltpu.SemaphoreType.DMA,
        ],
    )
    return kern(ids_w, word_emb, pos_emb, ln_gamma, ln_beta)


def kernel(input_ids, word_emb, pos_emb, ln_gamma, ln_beta):
    ids_w = (
        input_ids.T.reshape(NW, PPW, BATCH)
        .transpose(0, 2, 1)
        .reshape(NW, BATCH * PPW)
    )
    return _run(ids_w, word_emb, pos_emb, ln_gamma, ln_beta)


# speedup vs baseline: 1.0157x; 1.0157x over previous
import functools

import jax
import jax.numpy as jnp
from jax import lax
from jax.experimental import pallas as pl
from jax.experimental.pallas import tpu as pltpu
from jax.experimental.pallas import tpu_sc as plsc

VOCAB = 30522
HIDDEN = 768
BATCH = 64
SEQ = 512
EPS = 1e-12

NC = 2
NS = 16
NW = NC * NS
PPW = SEQ // NW
NJ = HIDDEN // 16
NT = NJ // 2

def _lanesum16(v):
    idx = lax.iota(jnp.int32, 16)
    dnums = lax.GatherDimensionNumbers(
        offset_dims=(), collapsed_slice_dims=(0,), start_index_map=(0,))
    for sh in (8, 4, 2, 1):
        perm = (idx ^ sh)[:, None]
        v = v + lax.gather(v, perm, dimension_numbers=dnums, slice_sizes=(1,),
                           unique_indices=True,
                           mode=lax.GatherScatterMode.PROMISE_IN_BOUNDS)
    return v


def _rsqrt16(v):
    x2 = v * 0.5
    i = lax.bitcast_convert_type(v, jnp.int32)
    i = jnp.int32(0x5F3759DF) - (i >> 1)
    y = lax.bitcast_convert_type(i, jnp.float32)
    y = y * (1.5 - x2 * y * y)
    y = y * (1.5 - x2 * y * y)
    y = y * (1.5 - x2 * y * y)
    y = y * (1.5 - x2 * y * y)
    return y


def _unpack2(v):
    a = lax.bitcast_convert_type(v << 16, jnp.float32)
    b = lax.bitcast_convert_type(v & jnp.int32(-65536), jnp.float32)
    return a, b


def _sc_body(ids_w, wemb, pos_pack, gb_pack, out,
             idx_v, pos_v, gb_v,
             in_a, in_b, out_a, out_b,
             gsem_a, gsem_b, ssem_a, ssem_b):
    c = lax.axis_index("c")
    s = lax.axis_index("s")
    w = s * NC + c

    pltpu.sync_copy(ids_w.at[w], idx_v)
    pltpu.sync_copy(pos_pack.at[pl.ds(w * PPW, PPW)], pos_v)
    pltpu.sync_copy(gb_pack, gb_v)

    def gather(b, buf, sem):
        pltpu.async_copy(wemb.at[idx_v.at[pl.ds(b * PPW, PPW)]], buf, sem)

    def compute(xin, xout):
        @plsc.parallel_loop(0, PPW, 1, unroll=2)
        def row_body(r):
            acc = [jnp.zeros((16,), jnp.float32) for _ in range(4)]
            accq = [jnp.zeros((16,), jnp.float32) for _ in range(4)]
            for t in range(NT):
                pa, pb = _unpack2(pos_v[r, t])
                x0 = xin[r, pl.ds(t * 32, 16)] + pa
                x1 = xin[r, pl.ds(t * 32 + 16, 16)] + pb
                k = 2 * (t % 2)
                acc[k] = acc[k] + x0
                acc[k + 1] = acc[k + 1] + x1
                accq[k] = accq[k] + x0 * x0
                accq[k + 1] = accq[k + 1] + x1 * x1
            sum_v = (acc[0] + acc[1]) + (acc[2] + acc[3])
            sq_v = (accq[0] + accq[1]) + (accq[2] + accq[3])
            mean_v = _lanesum16(sum_v) * (1.0 / HIDDEN)
            msq_v = _lanesum16(sq_v) * (1.0 / HIDDEN)
            var_v = msq_v - mean_v * mean_v
            inv_v = _rsqrt16(var_v + EPS)
            for t in range(NT):
                pa, pb = _unpack2(pos_v[r, t])
                for half, p in ((0, pa), (1, pb)):
                    sl = pl.ds(t * 32 + 16 * half, 16)
                    gv, bv = _unpack2(gb_v[2 * t + half])
                    x = xin[r, sl] + p
                    xout[r, sl] = (x - mean_v) * inv_v * gv + bv

    def halfstep(i, b, xin, xout, gsem, ssem):
        @pl.when(i > 0)
        def _():
            pltpu.make_async_copy(xout, out.at[0, pl.ds(w * PPW, PPW)], ssem).wait()

        pltpu.make_async_copy(wemb.at[pl.ds(0, PPW)], xin, gsem).wait()
        compute(xin, xout)

        @pl.when(b + 2 < BATCH)
        def _():
            gather(b + 2, xin, gsem)

        pltpu.async_copy(xout, out.at[b, pl.ds(w * PPW, PPW)], ssem)

    gather(0, in_a, gsem_a)
    gather(1, in_b, gsem_b)

    def loop_body(i, carry):
        halfstep(i, 2 * i, in_a, out_a, gsem_a, ssem_a)
        halfstep(i, 2 * i + 1, in_b, out_b, gsem_b, ssem_b)
        return carry

    lax.fori_loop(0, BATCH // 2, loop_body, 0)

    pltpu.make_async_copy(out_a, out.at[0, pl.ds(w * PPW, PPW)], ssem_a).wait()
    pltpu.make_async_copy(out_b, out.at[0, pl.ds(w * PPW, PPW)], ssem_b).wait()


@functools.partial(jax.jit, static_argnames=())
def _run(ids_w, word_emb, pos_pack, gb_pack):
    kern = pl.kernel(
        _sc_body,
        out_type=jax.ShapeDtypeStruct((BATCH, SEQ, HIDDEN), jnp.float32),
        mesh=plsc.VectorSubcoreMesh(core_axis_name="c", subcore_axis_name="s"),
        scratch_types=[
            pltpu.VMEM((BATCH * PPW,), jnp.int32),
            pltpu.VMEM((PPW, NT, 16), jnp.int32),
            pltpu.VMEM((NJ, 16), jnp.int32),
            pltpu.VMEM((PPW, HIDDEN), jnp.float32),
            pltpu.VMEM((PPW, HIDDEN), jnp.float32),
            pltpu.VMEM((PPW, HIDDEN), jnp.float32),
            pltpu.VMEM((PPW, HIDDEN), jnp.float32),
            pltpu.SemaphoreType.DMA,
            pltpu.SemaphoreType.DMA,
            pltpu.SemaphoreType.DMA,
            pltpu.SemaphoreType.DMA,
        ],
    )
    return kern(ids_w, word_emb, pos_pack, gb_pack)


def kernel(input_ids, word_emb, pos_emb, ln_gamma, ln_beta):
    ids_w = (
        input_ids.T.reshape(NW, PPW, BATCH)
        .transpose(0, 2, 1)
        .reshape(NW, BATCH * PPW)
    )
    def pack_pairs(a, b):
        au = lax.bitcast_convert_type(a.astype(jnp.bfloat16), jnp.uint16)
        bu = lax.bitcast_convert_type(b.astype(jnp.bfloat16), jnp.uint16)
        word = au.astype(jnp.uint32) | (bu.astype(jnp.uint32) << 16)
        return lax.bitcast_convert_type(word, jnp.int32)

    pr = pos_emb.reshape(SEQ, NT, 2, 16)
    pos_pack = pack_pairs(pr[:, :, 0, :], pr[:, :, 1, :])
    gb_pack = pack_pairs(ln_gamma.reshape(NJ, 16), ln_beta.reshape(NJ, 16))
    return _run(ids_w, word_emb, pos_pack, gb_pack)


# speedup vs baseline: 1.0549x; 1.0387x over previous
import functools

import jax
import jax.numpy as jnp
from jax import lax
from jax.experimental import pallas as pl
from jax.experimental.pallas import tpu as pltpu
from jax.experimental.pallas import tpu_sc as plsc

VOCAB = 30522
HIDDEN = 768
BATCH = 64
SEQ = 512
EPS = 1e-12

NC = 2
NS = 16
NW = NC * NS
PPW = SEQ // NW
NJ = HIDDEN // 16
NT = NJ // 2

def _lanesum16(v):
    idx = lax.iota(jnp.int32, 16)
    dnums = lax.GatherDimensionNumbers(
        offset_dims=(), collapsed_slice_dims=(0,), start_index_map=(0,))
    for sh in (8, 4, 2, 1):
        perm = (idx ^ sh)[:, None]
        v = v + lax.gather(v, perm, dimension_numbers=dnums, slice_sizes=(1,),
                           unique_indices=True,
                           mode=lax.GatherScatterMode.PROMISE_IN_BOUNDS)
    return v


def _rsqrt16(v):
    x2 = v * 0.5
    i = lax.bitcast_convert_type(v, jnp.int32)
    i = jnp.int32(0x5F3759DF) - (i >> 1)
    y = lax.bitcast_convert_type(i, jnp.float32)
    y = y * (1.5 - x2 * y * y)
    y = y * (1.5 - x2 * y * y)
    y = y * (1.5 - x2 * y * y)
    y = y * (1.5 - x2 * y * y)
    return y


def _unpack2(v):
    a = lax.bitcast_convert_type(v << 16, jnp.float32)
    b = lax.bitcast_convert_type(v & jnp.int32(-65536), jnp.float32)
    return a, b


def _sc_body(ids_w, wemb, pos_pack, gb_pack, out,
             idx_v, pos_v, gb_v, stats_v,
             in_a, in_b, out_a, out_b,
             gsem_a, gsem_b, ssem_a, ssem_b):
    c = lax.axis_index("c")
    s = lax.axis_index("s")
    w = s * NC + c

    pltpu.sync_copy(ids_w.at[w], idx_v)
    pltpu.sync_copy(pos_pack.at[pl.ds(w * PPW, PPW)], pos_v)
    pltpu.sync_copy(gb_pack, gb_v)

    def gather(b, buf, sem):
        pltpu.async_copy(wemb.at[idx_v.at[pl.ds(b * PPW, PPW)]], buf, sem)

    RB = 4

    def compute(xin, xout):
        @plsc.parallel_loop(0, PPW, 1, unroll=2)
        def row_body(r):
            acc = [jnp.zeros((16,), jnp.float32) for _ in range(4)]
            accq = [jnp.zeros((16,), jnp.float32) for _ in range(4)]
            for t in range(NT):
                pa, pb = _unpack2(pos_v[r, t])
                x0 = xin[r, pl.ds(t * 32, 16)] + pa
                x1 = xin[r, pl.ds(t * 32 + 16, 16)] + pb
                k = 2 * (t % 2)
                acc[k] = acc[k] + x0
                acc[k + 1] = acc[k + 1] + x1
                accq[k] = accq[k] + x0 * x0
                accq[k + 1] = accq[k + 1] + x1 * x1
            sum_v = (acc[0] + acc[1]) + (acc[2] + acc[3])
            sq_v = (accq[0] + accq[1]) + (accq[2] + accq[3])
            mean_v = _lanesum16(sum_v) * (1.0 / HIDDEN)
            msq_v = _lanesum16(sq_v) * (1.0 / HIDDEN)
            var_v = msq_v - mean_v * mean_v
            stats_v[r, 0] = mean_v
            stats_v[r, 1] = _rsqrt16(var_v + EPS)

        for blk in range(PPW // RB):
            rs = [blk * RB + k for k in range(RB)]
            ms = [stats_v[rr, 0] for rr in rs]
            ivs = [stats_v[rr, 1] for rr in rs]

            @plsc.parallel_loop(0, NT, 1, unroll=2)
            def pair_body(t):
                gv0, bv0 = _unpack2(gb_v[2 * t])
                gv1, bv1 = _unpack2(gb_v[2 * t + 1])
                for k in range(RB):
                    rr = rs[k]
                    pa, pb = _unpack2(pos_v[rr, t])
                    sl0 = pl.ds(t * 32, 16)
                    sl1 = pl.ds(t * 32 + 16, 16)
                    x0 = xin[rr, sl0] + pa
                    x1 = xin[rr, sl1] + pb
                    xout[rr, sl0] = (x0 - ms[k]) * ivs[k] * gv0 + bv0
                    xout[rr, sl1] = (x1 - ms[k]) * ivs[k] * gv1 + bv1

    def halfstep(i, b, xin, xout, gsem, ssem):
        @pl.when(i > 0)
        def _():
            pltpu.make_async_copy(xout, out.at[0, pl.ds(w * PPW, PPW)], ssem).wait()

        pltpu.make_async_copy(wemb.at[pl.ds(0, PPW)], xin, gsem).wait()
        compute(xin, xout)

        @pl.when(b + 2 < BATCH)
        def _():
            gather(b + 2, xin, gsem)

        pltpu.async_copy(xout, out.at[b, pl.ds(w * PPW, PPW)], ssem)

    gather(0, in_a, gsem_a)
    gather(1, in_b, gsem_b)

    def loop_body(i, carry):
        halfstep(i, 2 * i, in_a, out_a, gsem_a, ssem_a)
        halfstep(i, 2 * i + 1, in_b, out_b, gsem_b, ssem_b)
        return carry

    lax.fori_loop(0, BATCH // 2, loop_body, 0)

    pltpu.make_async_copy(out_a, out.at[0, pl.ds(w * PPW, PPW)], ssem_a).wait()
    pltpu.make_async_copy(out_b, out.at[0, pl.ds(w * PPW, PPW)], ssem_b).wait()


@functools.partial(jax.jit, static_argnames=())
def _run(ids_w, word_emb, pos_pack, gb_pack):
    kern = pl.kernel(
        _sc_body,
        out_type=jax.ShapeDtypeStruct((BATCH, SEQ, HIDDEN), jnp.float32),
        mesh=plsc.VectorSubcoreMesh(core_axis_name="c", subcore_axis_name="s"),
        scratch_types=[
            pltpu.VMEM((BATCH * PPW,), jnp.int32),
            pltpu.VMEM((PPW, NT, 16), jnp.int32),
            pltpu.VMEM((NJ, 16), jnp.int32),
            pltpu.VMEM((PPW, 2, 16), jnp.float32),
            pltpu.VMEM((PPW, HIDDEN), jnp.float32),
            pltpu.VMEM((PPW, HIDDEN), jnp.float32),
            pltpu.VMEM((PPW, HIDDEN), jnp.float32),
            pltpu.VMEM((PPW, HIDDEN), jnp.float32),
            pltpu.SemaphoreType.DMA,
            pltpu.SemaphoreType.DMA,
            pltpu.SemaphoreType.DMA,
            pltpu.SemaphoreType.DMA,
        ],
    )
    return kern(ids_w, word_emb, pos_pack, gb_pack)


def kernel(input_ids, word_emb, pos_emb, ln_gamma, ln_beta):
    ids_w = (
        input_ids.T.reshape(NW, PPW, BATCH)
        .transpose(0, 2, 1)
        .reshape(NW, BATCH * PPW)
    )
    def pack_pairs(a, b):
        au = lax.bitcast_convert_type(a.astype(jnp.bfloat16), jnp.uint16)
        bu = lax.bitcast_convert_type(b.astype(jnp.bfloat16), jnp.uint16)
        word = au.astype(jnp.uint32) | (bu.astype(jnp.uint32) << 16)
        return lax.bitcast_convert_type(word, jnp.int32)

    pr = pos_emb.reshape(SEQ, NT, 2, 16)
    pos_pack = pack_pairs(pr[:, :, 0, :], pr[:, :, 1, :])
    gb_pack = pack_pairs(ln_gamma.reshape(NJ, 16), ln_beta.reshape(NJ, 16))
    return _run(ids_w, word_emb, pos_pack, gb_pack)


# speedup vs baseline: 1.5126x; 1.4338x over previous
import jax
import jax.numpy as jnp
from jax import lax
from jax.experimental import pallas as pl
from jax.experimental.pallas import tpu as pltpu
from jax.experimental.pallas import tpu_sc as plsc

VOCAB = 30522
HIDDEN = 768
BATCH = 64
SEQ = 512
EPS = 1e-12

NC = 2
NS = 16
NW = NC * NS
PPW = SEQ // NW
NCH = 2
BC = BATCH // NCH


def _sc_gather_body(ids_w, wemb, g,
                    idx_v, buf_a, buf_b,
                    gsem_a, gsem_b, ssem_a, ssem_b):
    c = lax.axis_index("c")
    s = lax.axis_index("s")
    w = s * NC + c

    pltpu.sync_copy(ids_w.at[w], idx_v)

    def gather(b, buf, sem):
        pltpu.async_copy(wemb.at[idx_v.at[pl.ds(b * PPW, PPW)]], buf, sem)

    def halfstep(i, b, buf, gsem, ssem):
        @pl.when(i > 0)
        def _():
            pltpu.make_async_copy(buf, g.at[0, pl.ds(w * PPW, PPW)], ssem).wait()

        pltpu.make_async_copy(wemb.at[pl.ds(0, PPW)], buf, gsem).wait()

        pltpu.async_copy(buf, g.at[b, pl.ds(w * PPW, PPW)], ssem)

        @pl.when(b + 2 < BC)
        def _():
            gather(b + 2, buf, gsem)

    gather(0, buf_a, gsem_a)
    gather(1, buf_b, gsem_b)

    def loop_body(i, carry):
        halfstep(i, 2 * i, buf_a, gsem_a, ssem_a)
        halfstep(i, 2 * i + 1, buf_b, gsem_b, ssem_b)
        return carry

    lax.fori_loop(0, BC // 2, loop_body, 0)

    pltpu.make_async_copy(buf_a, g.at[0, pl.ds(w * PPW, PPW)], ssem_a).wait()
    pltpu.make_async_copy(buf_b, g.at[0, pl.ds(w * PPW, PPW)], ssem_b).wait()


def _sc_gather(ids_wc, word_emb):
    kern = pl.kernel(
        _sc_gather_body,
        out_type=jax.ShapeDtypeStruct((BC, SEQ, HIDDEN), jnp.float32),
        mesh=plsc.VectorSubcoreMesh(core_axis_name="c", subcore_axis_name="s"),
        scratch_types=[
            pltpu.VMEM((BC * PPW,), jnp.int32),
            pltpu.VMEM((PPW, HIDDEN), jnp.float32),
            pltpu.VMEM((PPW, HIDDEN), jnp.float32),
            pltpu.SemaphoreType.DMA,
            pltpu.SemaphoreType.DMA,
            pltpu.SemaphoreType.DMA,
            pltpu.SemaphoreType.DMA,
        ],
    )
    return kern(ids_wc, word_emb)


def _ln_block(g_ref, pos_ref, gam_ref, bet_ref, prev_ref, out_ref):
    x = g_ref[0] + pos_ref[...]
    mean = jnp.mean(x, axis=-1, keepdims=True)
    cx = x - mean
    var = jnp.mean(cx * cx, axis=-1, keepdims=True)
    y = cx * lax.rsqrt(var + EPS)
    out_ref[0] = y * gam_ref[...] + bet_ref[...]


def _tc_ln(g, pos, gam2, bet2, prev, chunk_off):
    return pl.pallas_call(
        _ln_block,
        grid=(BC,),
        in_specs=[
            pl.BlockSpec((1, SEQ, HIDDEN), lambda b: (b, 0, 0)),
            pl.BlockSpec((SEQ, HIDDEN), lambda b: (0, 0)),
            pl.BlockSpec((1, HIDDEN), lambda b: (0, 0)),
            pl.BlockSpec((1, HIDDEN), lambda b: (0, 0)),
            pl.BlockSpec(memory_space=pltpu.MemorySpace.HBM),
        ],
        out_specs=pl.BlockSpec(
            (1, SEQ, HIDDEN), lambda b, _o=chunk_off: (b + _o, 0, 0)),
        out_shape=jax.ShapeDtypeStruct((BATCH, SEQ, HIDDEN), jnp.float32),
        input_output_aliases={4: 0},
    )(g, pos, gam2, bet2, prev)


@jax.jit
def _run(ids_w, word_emb, pos_emb, gam2, bet2):
    out = jnp.zeros((BATCH, SEQ, HIDDEN), jnp.float32)
    for ch in range(NCH):
        g = _sc_gather(ids_w[ch], word_emb)
        out = _tc_ln(g, pos_emb, gam2, bet2, out, ch * BC)
    return out


def kernel(input_ids, word_emb, pos_emb, ln_gamma, ln_beta):
    ids_w = (
        input_ids.reshape(NCH, BC, NW, PPW)
        .transpose(0, 2, 1, 3)
        .reshape(NCH, NW, BC * PPW)
    )
    return _run(ids_w, word_emb, pos_emb,
                ln_gamma.reshape(1, HIDDEN), ln_beta.reshape(1, HIDDEN))


# speedup vs baseline: 1.5221x; 1.0063x over previous
import jax
import jax.numpy as jnp
from jax import lax
from jax.experimental import pallas as pl
from jax.experimental.pallas import tpu as pltpu
from jax.experimental.pallas import tpu_sc as plsc

VOCAB = 30522
HIDDEN = 768
BATCH = 64
SEQ = 512
EPS = 1e-12

NC = 2
NS = 16
NW = NC * NS
PPW = SEQ // NW
NCH = 1
BC = BATCH // NCH


def _sc_gather_body(ids_w, wemb, g,
                    idx_v, b0, b1, b2, b3,
                    g0, g1, g2, g3, s0, s1, s2, s3):
    c = lax.axis_index("c")
    s = lax.axis_index("s")
    w = s * NC + c

    pltpu.sync_copy(ids_w.at[w], idx_v)

    bufs = [b0, b1, b2, b3]
    gsems = [g0, g1, g2, g3]
    ssems = [s0, s1, s2, s3]

    def gather(b, k):
        pltpu.async_copy(wemb.at[idx_v.at[pl.ds(b * PPW, PPW)]],
                         bufs[k], gsems[k])

    def step(i, b, k):
        kn = (k + 2) % 4
        pltpu.make_async_copy(wemb.at[pl.ds(0, PPW)], bufs[k], gsems[k]).wait()
        pltpu.async_copy(bufs[k], g.at[b, pl.ds(w * PPW, PPW)], ssems[k])

        @pl.when(b >= 2)
        def _():
            pltpu.make_async_copy(
                bufs[kn], g.at[0, pl.ds(w * PPW, PPW)], ssems[kn]).wait()

        @pl.when(b + 2 < BC)
        def _():
            gather(b + 2, kn)

    gather(0, 0)
    gather(1, 1)

    def loop_body(i, carry):
        step(i, 4 * i, 0)
        step(i, 4 * i + 1, 1)
        step(i, 4 * i + 2, 2)
        step(i, 4 * i + 3, 3)
        return carry

    lax.fori_loop(0, BC // 4, loop_body, 0)

    pltpu.make_async_copy(b2, g.at[0, pl.ds(w * PPW, PPW)], s2).wait()
    pltpu.make_async_copy(b3, g.at[0, pl.ds(w * PPW, PPW)], s3).wait()


def _sc_gather(ids_wc, word_emb):
    kern = pl.kernel(
        _sc_gather_body,
        out_type=jax.ShapeDtypeStruct((BC, SEQ, HIDDEN), jnp.float32),
        mesh=plsc.VectorSubcoreMesh(core_axis_name="c", subcore_axis_name="s"),
        scratch_types=[
            pltpu.VMEM((BC * PPW,), jnp.int32),
            pltpu.VMEM((PPW, HIDDEN), jnp.float32),
            pltpu.VMEM((PPW, HIDDEN), jnp.float32),
            pltpu.VMEM((PPW, HIDDEN), jnp.float32),
            pltpu.VMEM((PPW, HIDDEN), jnp.float32),
            pltpu.SemaphoreType.DMA,
            pltpu.SemaphoreType.DMA,
            pltpu.SemaphoreType.DMA,
            pltpu.SemaphoreType.DMA,
            pltpu.SemaphoreType.DMA,
            pltpu.SemaphoreType.DMA,
            pltpu.SemaphoreType.DMA,
            pltpu.SemaphoreType.DMA,
        ],
    )
    return kern(ids_wc, word_emb)


def _ln_block(g_ref, pos_ref, gam_ref, bet_ref, prev_ref, out_ref):
    x = g_ref[0] + pos_ref[...]
    mean = jnp.mean(x, axis=-1, keepdims=True)
    cx = x - mean
    var = jnp.mean(cx * cx, axis=-1, keepdims=True)
    y = cx * lax.rsqrt(var + EPS)
    out_ref[0] = y * gam_ref[...] + bet_ref[...]


def _tc_ln(g, pos, gam2, bet2, prev, chunk_off):
    return pl.pallas_call(
        _ln_block,
        grid=(BC,),
        in_specs=[
            pl.BlockSpec((1, SEQ, HIDDEN), lambda b: (b, 0, 0)),
            pl.BlockSpec((SEQ, HIDDEN), lambda b: (0, 0)),
            pl.BlockSpec((1, HIDDEN), lambda b: (0, 0)),
            pl.BlockSpec((1, HIDDEN), lambda b: (0, 0)),
            pl.BlockSpec(memory_space=pltpu.MemorySpace.HBM),
        ],
        out_specs=pl.BlockSpec(
            (1, SEQ, HIDDEN), lambda b, _o=chunk_off: (b + _o, 0, 0)),
        out_shape=jax.ShapeDtypeStruct((BATCH, SEQ, HIDDEN), jnp.float32),
        input_output_aliases={4: 0},
    )(g, pos, gam2, bet2, prev)


@jax.jit
def _run(ids_w, word_emb, pos_emb, gam2, bet2):
    out = jnp.zeros((BATCH, SEQ, HIDDEN), jnp.float32)
    for ch in range(NCH):
        g = _sc_gather(ids_w[ch], word_emb)
        out = _tc_ln(g, pos_emb, gam2, bet2, out, ch * BC)
    return out


def kernel(input_ids, word_emb, pos_emb, ln_gamma, ln_beta):
    ids_w = (
        input_ids.reshape(NCH, BC, NW, PPW)
        .transpose(0, 2, 1, 3)
        .reshape(NCH, NW, BC * PPW)
    )
    return _run(ids_w, word_emb, pos_emb,
                ln_gamma.reshape(1, HIDDEN), ln_beta.reshape(1, HIDDEN))
